# Initial kernel scaffold; baseline (speedup 1.0000x reference)
import jax
import jax.numpy as jnp
from jax.experimental import pallas as pl


def kernel(x, edge_index, batch, W1_0, b1_0, W2_0, b2_0, g_0, be_0, W1_1, b1_1, W2_1, b2_1, g_1, be_1, W1_2, b1_2, W2_2, b2_2, g_2, be_2):
    raise NotImplementedError("write your pallas kernel here")


# speedup vs baseline: 6.1110x; 6.1110x over previous
import functools

import jax
import jax.numpy as jnp
from jax import lax
from jax.experimental import pallas as pl
from jax.experimental.pallas import tpu as pltpu
from jax.experimental.pallas import tpu_sc as plsc

_N = 50000
_E = 800000
_G = 64
_NP = 50048
_RPT = _NP // 16

_K = 3
_CHUNK = _K * 128
_CPT = 132
_EPT = _CPT * _CHUNK
_EP = 16 * _EPT
_IROWS_PT = _CPT * _K

_BM = 1088
_NB = _NP // _BM
_EPS = 1e-5


def _sc_agg(xt, src2, dst2):
    mesh = plsc.VectorSubcoreMesh(core_axis_name="c", subcore_axis_name="s")

    @functools.partial(
        pl.kernel,
        out_type=jax.ShapeDtypeStruct((2 * _NP, 32), jnp.float32),
        mesh=mesh,
        scratch_types=[
            pltpu.VMEM((2, _K, 128), jnp.int32),
            pltpu.VMEM((2, _K, 128), jnp.int32),
            pltpu.VMEM((2, _CHUNK, 32), jnp.float32),
            pltpu.VMEM_SHARED((_NP, 32), jnp.float32),
            pltpu.SemaphoreType.DMA,
            pltpu.SemaphoreType.DMA,
        ],
        compiler_params=pltpu.CompilerParams(use_tc_tiling_on_sc=False),
    )
    def agg(xt_hbm, src_hbm, dst_hbm, out_hbm, sidx, didx, rows, acc, g0, g1):
        c = lax.axis_index("c")
        s = lax.axis_index("s")
        gsems = (g0, g1)

        row0 = s * _RPT
        pltpu.sync_copy(xt_hbm.at[pl.ds(c * _NP + row0, _RPT)],
                        acc.at[pl.ds(row0, _RPT)])
        plsc.subcore_barrier()

        ebase = s * _IROWS_PT

        def fetch(k, buf):
            irow = ebase + k * _K
            pltpu.sync_copy(src_hbm.at[c, pl.ds(irow, _K)], sidx.at[buf])
            pltpu.sync_copy(dst_hbm.at[pl.ds(irow, _K)], didx.at[buf])
            for j in range(_K):
                pltpu.async_copy(xt_hbm.at[sidx.at[buf, j]],
                                 rows.at[buf, pl.ds(j * 128, 128)],
                                 gsems[buf])

        fetch(0, 0)

        def outer(i, carry):
            for b in range(2):
                k = i * 2 + b
                nb = 1 - b

                @pl.when(k < _CPT - 1)
                def _():
                    fetch(k + 1, nb)

                pltpu.make_async_copy(xt_hbm.at[pl.ds(0, _CHUNK)],
                                      rows.at[b], gsems[b]).wait()
                for j in range(_K):
                    pltpu.sync_copy(rows.at[b, pl.ds(j * 128, 128)],
                                    acc.at[didx.at[b, j]], add=True)
            return carry

        lax.fori_loop(0, _CPT // 2, outer, 0)
        plsc.subcore_barrier()

        pltpu.sync_copy(acc.at[pl.ds(row0, _RPT)],
                        out_hbm.at[pl.ds(c * _NP + row0, _RPT)])

    return agg(xt, src2, dst2)


def _tc_mlp_stats(hh, W1, b1, W2, b2):
    do = W2.shape[1]

    def body(hh_ref, w1_ref, b1_ref, w2_ref, b2_ref, y_ref, st_ref, sacc):
        i = pl.program_id(0)
        h = jnp.concatenate([hh_ref[0], hh_ref[1]], axis=1)
        t = jnp.maximum(
            jnp.dot(h, w1_ref[...], preferred_element_type=jnp.float32)
            + b1_ref[0, :], 0.0)
        y = jnp.dot(t, w2_ref[...], preferred_element_type=jnp.float32) \
            + b2_ref[0, :]
        y_ref[...] = y
        rid = i * _BM + lax.broadcasted_iota(jnp.int32, (_BM, 1), 0)
        ym = jnp.where(rid < _N, y, 0.0)
        upd = jnp.concatenate(
            [jnp.sum(ym, axis=0)[None, :], jnp.sum(ym * ym, axis=0)[None, :]],
            axis=0)
        prev = jnp.where(i == 0, jnp.zeros_like(upd), sacc[...])
        sacc[...] = prev + upd

        @pl.when(i == _NB - 1)
        def _():
            st_ref[...] = sacc[...]

    return pl.pallas_call(
        body,
        grid=(_NB,),
        in_specs=[
            pl.BlockSpec((2, _BM, 32), lambda i: (0, i, 0)),
            pl.BlockSpec((64, 64), lambda i: (0, 0)),
            pl.BlockSpec((1, 64), lambda i: (0, 0)),
            pl.BlockSpec((64, do), lambda i: (0, 0)),
            pl.BlockSpec((1, do), lambda i: (0, 0)),
        ],
        out_specs=[
            pl.BlockSpec((_BM, do), lambda i: (i, 0)),
            pl.BlockSpec((2, do), lambda i: (0, 0)),
        ],
        out_shape=[
            jax.ShapeDtypeStruct((_NP, do), jnp.float32),
            jax.ShapeDtypeStruct((2, do), jnp.float32),
        ],
        scratch_shapes=[pltpu.VMEM((2, do), jnp.float32)],
        compiler_params=pltpu.CompilerParams(
            dimension_semantics=("arbitrary",)),
    )(hh, W1, b1.reshape(1, -1), W2, b2.reshape(1, -1))


def _bn_coeffs(st_ref, g_ref, be_ref):
    mu = st_ref[0, :] * (1.0 / _N)
    var = st_ref[1, :] * (1.0 / _N) - mu * mu
    scale = g_ref[0, :] * lax.rsqrt(var + _EPS)
    shift = be_ref[0, :] - mu * scale
    return scale, shift


def _tc_norm(y, st, g, be):
    def body(y_ref, st_ref, g_ref, be_ref, o_ref):
        scale, shift = _bn_coeffs(st_ref, g_ref, be_ref)
        yn = jnp.maximum(y_ref[...] * scale[None, :] + shift[None, :], 0.0)
        o_ref[0, :, :] = yn[:, :32]
        o_ref[1, :, :] = yn[:, 32:]

    return pl.pallas_call(
        body,
        grid=(_NB,),
        in_specs=[
            pl.BlockSpec((_BM, 64), lambda i: (i, 0)),
            pl.BlockSpec((2, 64), lambda i: (0, 0)),
            pl.BlockSpec((1, 64), lambda i: (0, 0)),
            pl.BlockSpec((1, 64), lambda i: (0, 0)),
        ],
        out_specs=pl.BlockSpec((2, _BM, 32), lambda i: (0, i, 0)),
        out_shape=jax.ShapeDtypeStruct((2, _NP, 32), jnp.float32),
        compiler_params=pltpu.CompilerParams(
            dimension_semantics=("arbitrary",)),
    )(y, st, g.reshape(1, -1), be.reshape(1, -1))


def _tc_norm_pool(y, st, g, be, batch3):
    def body(y_ref, st_ref, g_ref, be_ref, b_ref, o_ref, pacc, cacc):
        i = pl.program_id(0)
        scale, shift = _bn_coeffs(st_ref, g_ref, be_ref)
        yn = jnp.maximum(y_ref[...] * scale[None, :] + shift[None, :], 0.0)
        bids = b_ref[0]
        oh = (bids == lax.broadcasted_iota(jnp.int32, (_G, _BM), 0)
              ).astype(jnp.float32)
        ps = jnp.dot(oh, yn, preferred_element_type=jnp.float32)
        cs = jnp.sum(oh, axis=1, keepdims=True)
        pprev = jnp.where(i == 0, jnp.zeros_like(ps), pacc[...])
        cprev = jnp.where(i == 0, jnp.zeros_like(cs), cacc[...])
        pacc[...] = pprev + ps
        cacc[...] = cprev + cs

        @pl.when(i == _NB - 1)
        def _():
            o_ref[...] = pacc[...] / jnp.maximum(cacc[...], 1.0)

    return pl.pallas_call(
        body,
        grid=(_NB,),
        in_specs=[
            pl.BlockSpec((_BM, 32), lambda i: (i, 0)),
            pl.BlockSpec((2, 32), lambda i: (0, 0)),
            pl.BlockSpec((1, 32), lambda i: (0, 0)),
            pl.BlockSpec((1, 32), lambda i: (0, 0)),
            pl.BlockSpec((1, 1, _BM), lambda i: (i, 0, 0)),
        ],
        out_specs=pl.BlockSpec((_G, 32), lambda i: (0, 0)),
        out_shape=jax.ShapeDtypeStruct((_G, 32), jnp.float32),
        scratch_shapes=[
            pltpu.VMEM((_G, 32), jnp.float32),
            pltpu.VMEM((_G, 1), jnp.float32),
        ],
        compiler_params=pltpu.CompilerParams(
            dimension_semantics=("arbitrary",)),
    )(y, st, g.reshape(1, -1), be.reshape(1, -1), batch3)


def kernel(x, edge_index, batch,
           W1_0, b1_0, W2_0, b2_0, g_0, be_0,
           W1_1, b1_1, W2_1, b2_1, g_1, be_1,
           W1_2, b1_2, W2_2, b2_2, g_2, be_2):
    params = [(W1_0, b1_0, W2_0, b2_0, g_0, be_0),
              (W1_1, b1_1, W2_1, b2_1, g_1, be_1),
              (W1_2, b1_2, W2_2, b2_2, g_2, be_2)]

    npad = _EP - _E
    src_p = jnp.concatenate([edge_index[0],
                             jnp.zeros((npad,), jnp.int32)])
    dst_p = jnp.concatenate([edge_index[1],
                             _N + (jnp.arange(npad, dtype=jnp.int32)
                                   % (_NP - _N))])
    src2 = jnp.stack([src_p, src_p + _NP]).reshape(2, _EP // 128, 128)
    dst2 = dst_p.reshape(_EP // 128, 128)
    batch3 = jnp.pad(batch, (0, _NP - _N),
                     constant_values=_G).reshape(_NB, 1, _BM)

    xt = jnp.pad(jnp.stack([x[:, :32], x[:, 32:]]),
                 ((0, 0), (0, _NP - _N), (0, 0))).reshape(2 * _NP, 32)

    out = None
    for l in range(3):
        W1, b1, W2, b2, g, be = params[l]
        hh = _sc_agg(xt, src2, dst2).reshape(2, _NP, 32)
        y, st = _tc_mlp_stats(hh, W1, b1, W2, b2)
        if l < 2:
            xt = _tc_norm(y, st, g, be).reshape(2 * _NP, 32)
        else:
            out = _tc_norm_pool(y, st, g, be, batch3)
    return out


# speedup vs baseline: 6.5348x; 1.0694x over previous
import functools

import jax
import jax.numpy as jnp
from jax import lax
from jax.experimental import pallas as pl
from jax.experimental.pallas import tpu as pltpu
from jax.experimental.pallas import tpu_sc as plsc

_N = 50000
_E = 800000
_G = 64
_NP = 50048
_RPT = _NP // 16

_K = 3
_CHUNK = _K * 128
_CPT = 132
_EPT = _CPT * _CHUNK
_EP = 16 * _EPT
_IROWS_PT = _CPT * _K

_BM = 1088
_NB = _NP // _BM
_EPS = 1e-5


def _sc_agg(xt, src2, dst2):
    mesh = plsc.VectorSubcoreMesh(core_axis_name="c", subcore_axis_name="s")

    @functools.partial(
        pl.kernel,
        out_type=jax.ShapeDtypeStruct((2 * _NP, 32), jnp.float32),
        mesh=mesh,
        scratch_types=[
            pltpu.VMEM((2, _K, 128), jnp.int32),
            pltpu.VMEM((2, _K, 128), jnp.int32),
            pltpu.VMEM((2, _CHUNK, 32), jnp.float32),
            pltpu.VMEM_SHARED((_NP, 32), jnp.float32),
            pltpu.SemaphoreType.DMA,
            pltpu.SemaphoreType.DMA,
            pltpu.SemaphoreType.DMA,
            pltpu.SemaphoreType.DMA,
            pltpu.SemaphoreType.DMA,
            pltpu.SemaphoreType.DMA,
        ],
        compiler_params=pltpu.CompilerParams(use_tc_tiling_on_sc=False),
    )
    def agg(xt_hbm, src_hbm, dst_hbm, out_hbm, sidx, didx, rows, acc,
            g0, g1, i0, i1, s0, s1):
        c = lax.axis_index("c")
        s = lax.axis_index("s")
        gsems = (g0, g1)
        isems = (i0, i1)
        ssems = (s0, s1)

        row0 = s * _RPT
        pltpu.sync_copy(xt_hbm.at[pl.ds(c * _NP + row0, _RPT)],
                        acc.at[pl.ds(row0, _RPT)])
        plsc.subcore_barrier()

        ebase = s * _IROWS_PT

        def idx_descs(k, buf):
            irow = ebase + k * _K
            return (pltpu.make_async_copy(src_hbm.at[c, pl.ds(irow, _K)],
                                          sidx.at[buf], isems[buf]),
                    pltpu.make_async_copy(dst_hbm.at[pl.ds(irow, _K)],
                                          didx.at[buf], isems[buf]))

        def fire_gathers(buf):
            for j in range(_K):
                pltpu.async_copy(xt_hbm.at[sidx.at[buf, j]],
                                 rows.at[buf, pl.ds(j * 128, 128)],
                                 gsems[buf])

        def fire_scatters(buf):
            for j in range(_K):
                pltpu.async_copy(rows.at[buf, pl.ds(j * 128, 128)],
                                 acc.at[didx.at[buf, j]], ssems[buf],
                                 add=True)

        def drain(sem, buf):
            pltpu.make_async_copy(xt_hbm.at[pl.ds(0, _CHUNK)],
                                  rows.at[buf], sem).wait()

        da, db = idx_descs(0, 0)
        da.start()
        db.start()
        da.wait()
        db.wait()
        fire_gathers(0)

        def outer(i, carry):
            for b in range(2):
                k = i * 2 + b
                nb = 1 - b

                @pl.when(k >= 1)
                def _():
                    drain(ssems[nb], nb)

                @pl.when(k < _CPT - 1)
                def _():
                    d1, d2 = idx_descs(k + 1, nb)
                    d1.start()
                    d2.start()

                drain(gsems[b], b)
                fire_scatters(b)

                @pl.when(k < _CPT - 1)
                def _():
                    d1, d2 = idx_descs(k + 1, nb)
                    d1.wait()
                    d2.wait()
                    fire_gathers(nb)
            return carry

        lax.fori_loop(0, _CPT // 2, outer, 0)
        drain(ssems[1], 1)
        plsc.subcore_barrier()

        pltpu.sync_copy(acc.at[pl.ds(row0, _RPT)],
                        out_hbm.at[pl.ds(c * _NP + row0, _RPT)])

    return agg(xt, src2, dst2)


def _tc_mlp_stats(hh, W1, b1, W2, b2):
    do = W2.shape[1]

    def body(hh_ref, w1_ref, b1_ref, w2_ref, b2_ref, y_ref, st_ref, sacc):
        i = pl.program_id(0)
        h = jnp.concatenate([hh_ref[0], hh_ref[1]], axis=1)
        t = jnp.maximum(
            jnp.dot(h, w1_ref[...], preferred_element_type=jnp.float32)
            + b1_ref[0, :], 0.0)
        y = jnp.dot(t, w2_ref[...], preferred_element_type=jnp.float32) \
            + b2_ref[0, :]
        y_ref[...] = y
        rid = i * _BM + lax.broadcasted_iota(jnp.int32, (_BM, 1), 0)
        ym = jnp.where(rid < _N, y, 0.0)
        upd = jnp.concatenate(
            [jnp.sum(ym, axis=0)[None, :], jnp.sum(ym * ym, axis=0)[None, :]],
            axis=0)
        prev = jnp.where(i == 0, jnp.zeros_like(upd), sacc[...])
        sacc[...] = prev + upd

        @pl.when(i == _NB - 1)
        def _():
            st_ref[...] = sacc[...]

    return pl.pallas_call(
        body,
        grid=(_NB,),
        in_specs=[
            pl.BlockSpec((2, _BM, 32), lambda i: (0, i, 0)),
            pl.BlockSpec((64, 64), lambda i: (0, 0)),
            pl.BlockSpec((1, 64), lambda i: (0, 0)),
            pl.BlockSpec((64, do), lambda i: (0, 0)),
            pl.BlockSpec((1, do), lambda i: (0, 0)),
        ],
        out_specs=[
            pl.BlockSpec((_BM, do), lambda i: (i, 0)),
            pl.BlockSpec((2, do), lambda i: (0, 0)),
        ],
        out_shape=[
            jax.ShapeDtypeStruct((_NP, do), jnp.float32),
            jax.ShapeDtypeStruct((2, do), jnp.float32),
        ],
        scratch_shapes=[pltpu.VMEM((2, do), jnp.float32)],
        compiler_params=pltpu.CompilerParams(
            dimension_semantics=("arbitrary",)),
    )(hh, W1, b1.reshape(1, -1), W2, b2.reshape(1, -1))


def _bn_coeffs(st_ref, g_ref, be_ref):
    mu = st_ref[0, :] * (1.0 / _N)
    var = st_ref[1, :] * (1.0 / _N) - mu * mu
    scale = g_ref[0, :] * lax.rsqrt(var + _EPS)
    shift = be_ref[0, :] - mu * scale
    return scale, shift


def _tc_norm(y, st, g, be):
    def body(y_ref, st_ref, g_ref, be_ref, o_ref):
        scale, shift = _bn_coeffs(st_ref, g_ref, be_ref)
        yn = jnp.maximum(y_ref[...] * scale[None, :] + shift[None, :], 0.0)
        o_ref[0, :, :] = yn[:, :32]
        o_ref[1, :, :] = yn[:, 32:]

    return pl.pallas_call(
        body,
        grid=(_NB,),
        in_specs=[
            pl.BlockSpec((_BM, 64), lambda i: (i, 0)),
            pl.BlockSpec((2, 64), lambda i: (0, 0)),
            pl.BlockSpec((1, 64), lambda i: (0, 0)),
            pl.BlockSpec((1, 64), lambda i: (0, 0)),
        ],
        out_specs=pl.BlockSpec((2, _BM, 32), lambda i: (0, i, 0)),
        out_shape=jax.ShapeDtypeStruct((2, _NP, 32), jnp.float32),
        compiler_params=pltpu.CompilerParams(
            dimension_semantics=("arbitrary",)),
    )(y, st, g.reshape(1, -1), be.reshape(1, -1))


def _tc_norm_pool(y, st, g, be, batch3):
    def body(y_ref, st_ref, g_ref, be_ref, b_ref, o_ref, pacc, cacc):
        i = pl.program_id(0)
        scale, shift = _bn_coeffs(st_ref, g_ref, be_ref)
        yn = jnp.maximum(y_ref[...] * scale[None, :] + shift[None, :], 0.0)
        bids = b_ref[0]
        oh = (bids == lax.broadcasted_iota(jnp.int32, (_G, _BM), 0)
              ).astype(jnp.float32)
        ps = jnp.dot(oh, yn, preferred_element_type=jnp.float32)
        cs = jnp.sum(oh, axis=1, keepdims=True)
        pprev = jnp.where(i == 0, jnp.zeros_like(ps), pacc[...])
        cprev = jnp.where(i == 0, jnp.zeros_like(cs), cacc[...])
        pacc[...] = pprev + ps
        cacc[...] = cprev + cs

        @pl.when(i == _NB - 1)
        def _():
            o_ref[...] = pacc[...] / jnp.maximum(cacc[...], 1.0)

    return pl.pallas_call(
        body,
        grid=(_NB,),
        in_specs=[
            pl.BlockSpec((_BM, 32), lambda i: (i, 0)),
            pl.BlockSpec((2, 32), lambda i: (0, 0)),
            pl.BlockSpec((1, 32), lambda i: (0, 0)),
            pl.BlockSpec((1, 32), lambda i: (0, 0)),
            pl.BlockSpec((1, 1, _BM), lambda i: (i, 0, 0)),
        ],
        out_specs=pl.BlockSpec((_G, 32), lambda i: (0, 0)),
        out_shape=jax.ShapeDtypeStruct((_G, 32), jnp.float32),
        scratch_shapes=[
            pltpu.VMEM((_G, 32), jnp.float32),
            pltpu.VMEM((_G, 1), jnp.float32),
        ],
        compiler_params=pltpu.CompilerParams(
            dimension_semantics=("arbitrary",)),
    )(y, st, g.reshape(1, -1), be.reshape(1, -1), batch3)


def kernel(x, edge_index, batch,
           W1_0, b1_0, W2_0, b2_0, g_0, be_0,
           W1_1, b1_1, W2_1, b2_1, g_1, be_1,
           W1_2, b1_2, W2_2, b2_2, g_2, be_2):
    params = [(W1_0, b1_0, W2_0, b2_0, g_0, be_0),
              (W1_1, b1_1, W2_1, b2_1, g_1, be_1),
              (W1_2, b1_2, W2_2, b2_2, g_2, be_2)]

    npad = _EP - _E
    src_p = jnp.concatenate([edge_index[0],
                             jnp.zeros((npad,), jnp.int32)])
    dst_p = jnp.concatenate([edge_index[1],
                             _N + (jnp.arange(npad, dtype=jnp.int32)
                                   % (_NP - _N))])
    src2 = jnp.stack([src_p, src_p + _NP]).reshape(2, _EP // 128, 128)
    dst2 = dst_p.reshape(_EP // 128, 128)
    batch3 = jnp.pad(batch, (0, _NP - _N),
                     constant_values=_G).reshape(_NB, 1, _BM)

    xt = jnp.pad(jnp.stack([x[:, :32], x[:, 32:]]),
                 ((0, 0), (0, _NP - _N), (0, 0))).reshape(2 * _NP, 32)

    out = None
    for l in range(3):
        W1, b1, W2, b2, g, be = params[l]
        hh = _sc_agg(xt, src2, dst2).reshape(2, _NP, 32)
        y, st = _tc_mlp_stats(hh, W1, b1, W2, b2)
        if l < 2:
            xt = _tc_norm(y, st, g, be).reshape(2 * _NP, 32)
        else:
            out = _tc_norm_pool(y, st, g, be, batch3)
    return out


# speedup vs baseline: 7.2133x; 1.1038x over previous
import functools

import jax
import jax.numpy as jnp
from jax import lax
from jax.experimental import pallas as pl
from jax.experimental.pallas import tpu as pltpu
from jax.experimental.pallas import tpu_sc as plsc

_N = 50000
_E = 800000
_G = 64
_NP = 50048
_RPT = _NP // 16

_K = 3
_CHUNK = _K * 128
_CPT = 132
_EPT = _CPT * _CHUNK
_EP = 16 * _EPT
_IROWS_PT = _CPT * _K

_BM = 6256
_NB = _NP // _BM
_EPS = 1e-5


def _sc_agg(xt, src2, dst2):
    mesh = plsc.VectorSubcoreMesh(core_axis_name="c", subcore_axis_name="s")

    @functools.partial(
        pl.kernel,
        out_type=jax.ShapeDtypeStruct((2, _NP, 32), jnp.float32),
        mesh=mesh,
        scratch_types=[
            pltpu.VMEM((2, _K, 128), jnp.int32),
            pltpu.VMEM((2, _K, 128), jnp.int32),
            pltpu.VMEM((2, _CHUNK, 32), jnp.float32),
            pltpu.VMEM_SHARED((_NP, 32), jnp.float32),
            pltpu.SemaphoreType.DMA,
            pltpu.SemaphoreType.DMA,
            pltpu.SemaphoreType.DMA,
            pltpu.SemaphoreType.DMA,
            pltpu.SemaphoreType.DMA,
            pltpu.SemaphoreType.DMA,
        ],
        compiler_params=pltpu.CompilerParams(use_tc_tiling_on_sc=False),
    )
    def agg(xt_hbm, src_hbm, dst_hbm, out_hbm, sidx, didx, rows, acc,
            g0, g1, i0, i1, s0, s1):
        c = lax.axis_index("c")
        s = lax.axis_index("s")
        gsems = (g0, g1)
        isems = (i0, i1)
        ssems = (s0, s1)

        row0 = s * _RPT
        pltpu.sync_copy(xt_hbm.at[c, pl.ds(row0, _RPT)],
                        acc.at[pl.ds(row0, _RPT)])
        plsc.subcore_barrier()

        ebase = s * _IROWS_PT

        def idx_descs(k, buf):
            irow = ebase + k * _K
            return (pltpu.make_async_copy(src_hbm.at[pl.ds(irow, _K)],
                                          sidx.at[buf], isems[buf]),
                    pltpu.make_async_copy(dst_hbm.at[pl.ds(irow, _K)],
                                          didx.at[buf], isems[buf]))

        def fire_gathers(buf):
            for j in range(_K):
                pltpu.async_copy(xt_hbm.at[c].at[sidx.at[buf, j]],
                                 rows.at[buf, pl.ds(j * 128, 128)],
                                 gsems[buf])

        def fire_scatters(buf):
            for j in range(_K):
                pltpu.async_copy(rows.at[buf, pl.ds(j * 128, 128)],
                                 acc.at[didx.at[buf, j]], ssems[buf],
                                 add=True)

        def drain(sem, buf):
            pltpu.make_async_copy(xt_hbm.at[0, pl.ds(0, _CHUNK)],
                                  rows.at[buf], sem).wait()

        da, db = idx_descs(0, 0)
        da.start()
        db.start()
        da.wait()
        db.wait()
        fire_gathers(0)

        def outer(i, carry):
            for b in range(2):
                k = i * 2 + b
                nb = 1 - b

                @pl.when(k >= 1)
                def _():
                    drain(ssems[nb], nb)

                @pl.when(k < _CPT - 1)
                def _():
                    d1, d2 = idx_descs(k + 1, nb)
                    d1.start()
                    d2.start()

                drain(gsems[b], b)
                fire_scatters(b)

                @pl.when(k < _CPT - 1)
                def _():
                    d1, d2 = idx_descs(k + 1, nb)
                    d1.wait()
                    d2.wait()
                    fire_gathers(nb)
            return carry

        lax.fori_loop(0, _CPT // 2, outer, 0)
        drain(ssems[1], 1)
        plsc.subcore_barrier()

        pltpu.sync_copy(acc.at[pl.ds(row0, _RPT)],
                        out_hbm.at[c, pl.ds(row0, _RPT)])

    return agg(xt, src2, dst2)


def _tc_mlp_stats(hh, W1, b1, W2, b2):
    do = W2.shape[1]

    def body(hh_ref, w1_ref, b1_ref, w2_ref, b2_ref, y_ref, st_ref, sacc):
        i = pl.program_id(0)
        h = jnp.concatenate([hh_ref[0], hh_ref[1]], axis=1)
        t = jnp.maximum(
            jnp.dot(h, w1_ref[...], preferred_element_type=jnp.float32)
            + b1_ref[0, :], 0.0)
        y = jnp.dot(t, w2_ref[...], preferred_element_type=jnp.float32) \
            + b2_ref[0, :]
        y_ref[...] = y
        rid = i * _BM + lax.broadcasted_iota(jnp.int32, (_BM, 1), 0)
        ym = jnp.where(rid < _N, y, 0.0)
        upd = jnp.concatenate(
            [jnp.sum(ym, axis=0)[None, :], jnp.sum(ym * ym, axis=0)[None, :]],
            axis=0)
        prev = jnp.where(i == 0, jnp.zeros_like(upd), sacc[...])
        sacc[...] = prev + upd

        @pl.when(i == _NB - 1)
        def _():
            st_ref[...] = sacc[...]

    return pl.pallas_call(
        body,
        grid=(_NB,),
        in_specs=[
            pl.BlockSpec((2, _BM, 32), lambda i: (0, i, 0)),
            pl.BlockSpec((64, 64), lambda i: (0, 0)),
            pl.BlockSpec((1, 64), lambda i: (0, 0)),
            pl.BlockSpec((64, do), lambda i: (0, 0)),
            pl.BlockSpec((1, do), lambda i: (0, 0)),
        ],
        out_specs=[
            pl.BlockSpec((_BM, do), lambda i: (i, 0)),
            pl.BlockSpec((2, do), lambda i: (0, 0)),
        ],
        out_shape=[
            jax.ShapeDtypeStruct((_NP, do), jnp.float32),
            jax.ShapeDtypeStruct((2, do), jnp.float32),
        ],
        scratch_shapes=[pltpu.VMEM((2, do), jnp.float32)],
        compiler_params=pltpu.CompilerParams(
            dimension_semantics=("arbitrary",)),
    )(hh, W1, b1.reshape(1, -1), W2, b2.reshape(1, -1))


def _bn_coeffs(st_ref, g_ref, be_ref):
    mu = st_ref[0, :] * (1.0 / _N)
    var = st_ref[1, :] * (1.0 / _N) - mu * mu
    scale = g_ref[0, :] * lax.rsqrt(var + _EPS)
    shift = be_ref[0, :] - mu * scale
    return scale, shift


def _tc_norm(y, st, g, be):
    def body(y_ref, st_ref, g_ref, be_ref, o_ref):
        scale, shift = _bn_coeffs(st_ref, g_ref, be_ref)
        yn = jnp.maximum(y_ref[...] * scale[None, :] + shift[None, :], 0.0)
        o_ref[0, :, :] = yn[:, :32]
        o_ref[1, :, :] = yn[:, 32:]

    return pl.pallas_call(
        body,
        grid=(_NB,),
        in_specs=[
            pl.BlockSpec((_BM, 64), lambda i: (i, 0)),
            pl.BlockSpec((2, 64), lambda i: (0, 0)),
            pl.BlockSpec((1, 64), lambda i: (0, 0)),
            pl.BlockSpec((1, 64), lambda i: (0, 0)),
        ],
        out_specs=pl.BlockSpec((2, _BM, 32), lambda i: (0, i, 0)),
        out_shape=jax.ShapeDtypeStruct((2, _NP, 32), jnp.float32),
        compiler_params=pltpu.CompilerParams(
            dimension_semantics=("arbitrary",)),
    )(y, st, g.reshape(1, -1), be.reshape(1, -1))


def _tc_norm_pool(y, st, g, be, batch3):
    def body(y_ref, st_ref, g_ref, be_ref, b_ref, o_ref, pacc, cacc):
        i = pl.program_id(0)
        scale, shift = _bn_coeffs(st_ref, g_ref, be_ref)
        yn = jnp.maximum(y_ref[...] * scale[None, :] + shift[None, :], 0.0)
        bids = b_ref[0]
        oh = (bids == lax.broadcasted_iota(jnp.int32, (_G, _BM), 0)
              ).astype(jnp.float32)
        ps = jnp.dot(oh, yn, preferred_element_type=jnp.float32)
        cs = jnp.sum(oh, axis=1, keepdims=True)
        pprev = jnp.where(i == 0, jnp.zeros_like(ps), pacc[...])
        cprev = jnp.where(i == 0, jnp.zeros_like(cs), cacc[...])
        pacc[...] = pprev + ps
        cacc[...] = cprev + cs

        @pl.when(i == _NB - 1)
        def _():
            o_ref[...] = pacc[...] / jnp.maximum(cacc[...], 1.0)

    return pl.pallas_call(
        body,
        grid=(_NB,),
        in_specs=[
            pl.BlockSpec((_BM, 32), lambda i: (i, 0)),
            pl.BlockSpec((2, 32), lambda i: (0, 0)),
            pl.BlockSpec((1, 32), lambda i: (0, 0)),
            pl.BlockSpec((1, 32), lambda i: (0, 0)),
            pl.BlockSpec((1, 1, _BM), lambda i: (i, 0, 0)),
        ],
        out_specs=pl.BlockSpec((_G, 32), lambda i: (0, 0)),
        out_shape=jax.ShapeDtypeStruct((_G, 32), jnp.float32),
        scratch_shapes=[
            pltpu.VMEM((_G, 32), jnp.float32),
            pltpu.VMEM((_G, 1), jnp.float32),
        ],
        compiler_params=pltpu.CompilerParams(
            dimension_semantics=("arbitrary",)),
    )(y, st, g.reshape(1, -1), be.reshape(1, -1), batch3)


def kernel(x, edge_index, batch,
           W1_0, b1_0, W2_0, b2_0, g_0, be_0,
           W1_1, b1_1, W2_1, b2_1, g_1, be_1,
           W1_2, b1_2, W2_2, b2_2, g_2, be_2):
    params = [(W1_0, b1_0, W2_0, b2_0, g_0, be_0),
              (W1_1, b1_1, W2_1, b2_1, g_1, be_1),
              (W1_2, b1_2, W2_2, b2_2, g_2, be_2)]

    npad = _EP - _E
    src_p = jnp.concatenate([edge_index[0],
                             jnp.zeros((npad,), jnp.int32)])
    dst_p = jnp.concatenate([edge_index[1],
                             _N + (jnp.arange(npad, dtype=jnp.int32)
                                   % (_NP - _N))])
    src2 = src_p.reshape(_EP // 128, 128)
    dst2 = dst_p.reshape(_EP // 128, 128)
    batch3 = jnp.pad(batch, (0, _NP - _N),
                     constant_values=_G).reshape(_NB, 1, _BM)

    xt = jnp.pad(jnp.stack([x[:, :32], x[:, 32:]]),
                 ((0, 0), (0, _NP - _N), (0, 0)))

    out = None
    for l in range(3):
        W1, b1, W2, b2, g, be = params[l]
        hh = _sc_agg(xt, src2, dst2)
        y, st = _tc_mlp_stats(hh, W1, b1, W2, b2)
        if l < 2:
            xt = _tc_norm(y, st, g, be)
        else:
            out = _tc_norm_pool(y, st, g, be, batch3)
    return out


# speedup vs baseline: 7.2211x; 1.0011x over previous
import functools

import jax
import jax.numpy as jnp
from jax import lax
from jax.experimental import pallas as pl
from jax.experimental.pallas import tpu as pltpu
from jax.experimental.pallas import tpu_sc as plsc

_N = 50000
_E = 800000
_G = 64
_NP = 50048
_RPT = _NP // 16

_K = 3
_CHUNK = _K * 128
_CPT = 132
_EPT = _CPT * _CHUNK
_EP = 16 * _EPT
_IROWS_PT = _CPT * _K

_BM = 6256
_NB = _NP // _BM
_EPS = 1e-5


def _sc_agg(xt, src2, dst2):
    mesh = plsc.VectorSubcoreMesh(core_axis_name="c", subcore_axis_name="s")

    @functools.partial(
        pl.kernel,
        out_type=jax.ShapeDtypeStruct((2, _NP, 32), jnp.float32),
        mesh=mesh,
        scratch_types=[
            pltpu.VMEM((2, _CHUNK), jnp.int32),
            pltpu.VMEM((2, _K, 128), jnp.int32),
            pltpu.VMEM((2, _CHUNK, 32), jnp.float32),
            pltpu.VMEM_SHARED((_NP, 32), jnp.float32),
            pltpu.SemaphoreType.DMA,
            pltpu.SemaphoreType.DMA,
            pltpu.SemaphoreType.DMA,
            pltpu.SemaphoreType.DMA,
            pltpu.SemaphoreType.DMA,
            pltpu.SemaphoreType.DMA,
        ],
        compiler_params=pltpu.CompilerParams(use_tc_tiling_on_sc=False),
    )
    def agg(xt_hbm, src_hbm, dst_hbm, out_hbm, sidx, didx, rows, acc,
            g0, g1, i0, i1, s0, s1):
        c = lax.axis_index("c")
        s = lax.axis_index("s")
        gsems = (g0, g1)
        isems = (i0, i1)
        ssems = (s0, s1)

        row0 = s * _RPT
        pltpu.sync_copy(xt_hbm.at[c, pl.ds(row0, _RPT)],
                        acc.at[pl.ds(row0, _RPT)])
        plsc.subcore_barrier()

        ebase = s * _IROWS_PT

        def idx_descs(k, buf):
            irow = ebase + k * _K
            return (pltpu.make_async_copy(
                        src_hbm.at[pl.ds(irow * 128, _CHUNK)],
                        sidx.at[buf], isems[buf]),
                    pltpu.make_async_copy(dst_hbm.at[pl.ds(irow, _K)],
                                          didx.at[buf], isems[buf]))

        def fire_gathers(buf):
            pltpu.async_copy(xt_hbm.at[c].at[sidx.at[buf]],
                             rows.at[buf], gsems[buf])

        def fire_scatters(buf):
            for j in range(_K):
                pltpu.async_copy(rows.at[buf, pl.ds(j * 128, 128)],
                                 acc.at[didx.at[buf, j]], ssems[buf],
                                 add=True)

        def drain(sem, buf):
            pltpu.make_async_copy(xt_hbm.at[0, pl.ds(0, _CHUNK)],
                                  rows.at[buf], sem).wait()

        da, db = idx_descs(0, 0)
        da.start()
        db.start()
        da.wait()
        db.wait()
        fire_gathers(0)

        def outer(i, carry):
            for b in range(2):
                k = i * 2 + b
                nb = 1 - b

                @pl.when(k >= 1)
                def _():
                    drain(ssems[nb], nb)

                @pl.when(k < _CPT - 1)
                def _():
                    d1, d2 = idx_descs(k + 1, nb)
                    d1.start()
                    d2.start()

                drain(gsems[b], b)
                fire_scatters(b)

                @pl.when(k < _CPT - 1)
                def _():
                    d1, d2 = idx_descs(k + 1, nb)
                    d1.wait()
                    d2.wait()
                    fire_gathers(nb)
            return carry

        lax.fori_loop(0, _CPT // 2, outer, 0)
        drain(ssems[1], 1)
        plsc.subcore_barrier()

        pltpu.sync_copy(acc.at[pl.ds(row0, _RPT)],
                        out_hbm.at[c, pl.ds(row0, _RPT)])

    return agg(xt, src2, dst2)


def _tc_mlp_stats(hh, W1, b1, W2, b2):
    do = W2.shape[1]

    def body(hh_ref, w1_ref, b1_ref, w2_ref, b2_ref, y_ref, st_ref, sacc):
        i = pl.program_id(0)
        h = jnp.concatenate([hh_ref[0], hh_ref[1]], axis=1)
        t = jnp.maximum(
            jnp.dot(h, w1_ref[...], preferred_element_type=jnp.float32)
            + b1_ref[0, :], 0.0)
        y = jnp.dot(t, w2_ref[...], preferred_element_type=jnp.float32) \
            + b2_ref[0, :]
        y_ref[...] = y
        rid = i * _BM + lax.broadcasted_iota(jnp.int32, (_BM, 1), 0)
        ym = jnp.where(rid < _N, y, 0.0)
        upd = jnp.concatenate(
            [jnp.sum(ym, axis=0)[None, :], jnp.sum(ym * ym, axis=0)[None, :]],
            axis=0)
        prev = jnp.where(i == 0, jnp.zeros_like(upd), sacc[...])
        sacc[...] = prev + upd

        @pl.when(i == _NB - 1)
        def _():
            st_ref[...] = sacc[...]

    return pl.pallas_call(
        body,
        grid=(_NB,),
        in_specs=[
            pl.BlockSpec((2, _BM, 32), lambda i: (0, i, 0)),
            pl.BlockSpec((64, 64), lambda i: (0, 0)),
            pl.BlockSpec((1, 64), lambda i: (0, 0)),
            pl.BlockSpec((64, do), lambda i: (0, 0)),
            pl.BlockSpec((1, do), lambda i: (0, 0)),
        ],
        out_specs=[
            pl.BlockSpec((_BM, do), lambda i: (i, 0)),
            pl.BlockSpec((2, do), lambda i: (0, 0)),
        ],
        out_shape=[
            jax.ShapeDtypeStruct((_NP, do), jnp.float32),
            jax.ShapeDtypeStruct((2, do), jnp.float32),
        ],
        scratch_shapes=[pltpu.VMEM((2, do), jnp.float32)],
        compiler_params=pltpu.CompilerParams(
            dimension_semantics=("arbitrary",)),
    )(hh, W1, b1.reshape(1, -1), W2, b2.reshape(1, -1))


def _bn_coeffs(st_ref, g_ref, be_ref):
    mu = st_ref[0, :] * (1.0 / _N)
    var = st_ref[1, :] * (1.0 / _N) - mu * mu
    scale = g_ref[0, :] * lax.rsqrt(var + _EPS)
    shift = be_ref[0, :] - mu * scale
    return scale, shift


def _tc_norm(y, st, g, be):
    def body(y_ref, st_ref, g_ref, be_ref, o_ref):
        scale, shift = _bn_coeffs(st_ref, g_ref, be_ref)
        yn = jnp.maximum(y_ref[...] * scale[None, :] + shift[None, :], 0.0)
        o_ref[0, :, :] = yn[:, :32]
        o_ref[1, :, :] = yn[:, 32:]

    return pl.pallas_call(
        body,
        grid=(_NB,),
        in_specs=[
            pl.BlockSpec((_BM, 64), lambda i: (i, 0)),
            pl.BlockSpec((2, 64), lambda i: (0, 0)),
            pl.BlockSpec((1, 64), lambda i: (0, 0)),
            pl.BlockSpec((1, 64), lambda i: (0, 0)),
        ],
        out_specs=pl.BlockSpec((2, _BM, 32), lambda i: (0, i, 0)),
        out_shape=jax.ShapeDtypeStruct((2, _NP, 32), jnp.float32),
        compiler_params=pltpu.CompilerParams(
            dimension_semantics=("arbitrary",)),
    )(y, st, g.reshape(1, -1), be.reshape(1, -1))


def _tc_norm_pool(y, st, g, be, batch3):
    def body(y_ref, st_ref, g_ref, be_ref, b_ref, o_ref, pacc, cacc):
        i = pl.program_id(0)
        scale, shift = _bn_coeffs(st_ref, g_ref, be_ref)
        yn = jnp.maximum(y_ref[...] * scale[None, :] + shift[None, :], 0.0)
        bids = b_ref[0]
        oh = (bids == lax.broadcasted_iota(jnp.int32, (_G, _BM), 0)
              ).astype(jnp.float32)
        ps = jnp.dot(oh, yn, preferred_element_type=jnp.float32)
        cs = jnp.sum(oh, axis=1, keepdims=True)
        pprev = jnp.where(i == 0, jnp.zeros_like(ps), pacc[...])
        cprev = jnp.where(i == 0, jnp.zeros_like(cs), cacc[...])
        pacc[...] = pprev + ps
        cacc[...] = cprev + cs

        @pl.when(i == _NB - 1)
        def _():
            o_ref[...] = pacc[...] / jnp.maximum(cacc[...], 1.0)

    return pl.pallas_call(
        body,
        grid=(_NB,),
        in_specs=[
            pl.BlockSpec((_BM, 32), lambda i: (i, 0)),
            pl.BlockSpec((2, 32), lambda i: (0, 0)),
            pl.BlockSpec((1, 32), lambda i: (0, 0)),
            pl.BlockSpec((1, 32), lambda i: (0, 0)),
            pl.BlockSpec((1, 1, _BM), lambda i: (i, 0, 0)),
        ],
        out_specs=pl.BlockSpec((_G, 32), lambda i: (0, 0)),
        out_shape=jax.ShapeDtypeStruct((_G, 32), jnp.float32),
        scratch_shapes=[
            pltpu.VMEM((_G, 32), jnp.float32),
            pltpu.VMEM((_G, 1), jnp.float32),
        ],
        compiler_params=pltpu.CompilerParams(
            dimension_semantics=("arbitrary",)),
    )(y, st, g.reshape(1, -1), be.reshape(1, -1), batch3)


def kernel(x, edge_index, batch,
           W1_0, b1_0, W2_0, b2_0, g_0, be_0,
           W1_1, b1_1, W2_1, b2_1, g_1, be_1,
           W1_2, b1_2, W2_2, b2_2, g_2, be_2):
    params = [(W1_0, b1_0, W2_0, b2_0, g_0, be_0),
              (W1_1, b1_1, W2_1, b2_1, g_1, be_1),
              (W1_2, b1_2, W2_2, b2_2, g_2, be_2)]

    npad = _EP - _E
    src_p = jnp.concatenate([edge_index[0],
                             jnp.zeros((npad,), jnp.int32)])
    dst_p = jnp.concatenate([edge_index[1],
                             _N + (jnp.arange(npad, dtype=jnp.int32)
                                   % (_NP - _N))])
    src2 = src_p
    dst2 = dst_p.reshape(_EP // 128, 128)
    batch3 = jnp.pad(batch, (0, _NP - _N),
                     constant_values=_G).reshape(_NB, 1, _BM)

    xt = jnp.pad(jnp.stack([x[:, :32], x[:, 32:]]),
                 ((0, 0), (0, _NP - _N), (0, 0)))

    out = None
    for l in range(3):
        W1, b1, W2, b2, g, be = params[l]
        hh = _sc_agg(xt, src2, dst2)
        y, st = _tc_mlp_stats(hh, W1, b1, W2, b2)
        if l < 2:
            xt = _tc_norm(y, st, g, be)
        else:
            out = _tc_norm_pool(y, st, g, be, batch3)
    return out


# speedup vs baseline: 9.0633x; 1.2551x over previous
import functools

import jax
import jax.numpy as jnp
from jax import lax
from jax.experimental import pallas as pl
from jax.experimental.pallas import tpu as pltpu
from jax.experimental.pallas import tpu_sc as plsc

_N = 50000
_E = 800000
_G = 64
_NP = 50048
_RPT = _NP // 16
_NQ = _NP // 4
_NQR = _N // 4

_K = 3
_CHUNK = _K * 128
_CPT = 132
_EPT = _CPT * _CHUNK
_EP = 16 * _EPT
_IROWS_PT = _CPT * _K

_BP = 3128
_NBP = _NQ // _BP
_EPS = 1e-5


def _sc_agg(xt, src2, dst2):
    mesh = plsc.VectorSubcoreMesh(core_axis_name="c", subcore_axis_name="s")

    @functools.partial(
        pl.kernel,
        out_type=jax.ShapeDtypeStruct((2, _NP, 32), jnp.float32),
        mesh=mesh,
        scratch_types=[
            pltpu.VMEM((2, _K, 128), jnp.int32),
            pltpu.VMEM((2, _K, 128), jnp.int32),
            pltpu.VMEM((2, _CHUNK, 32), jnp.float32),
            pltpu.VMEM_SHARED((_NP, 32), jnp.float32),
            pltpu.SemaphoreType.DMA,
            pltpu.SemaphoreType.DMA,
            pltpu.SemaphoreType.DMA,
            pltpu.SemaphoreType.DMA,
            pltpu.SemaphoreType.DMA,
            pltpu.SemaphoreType.DMA,
        ],
        compiler_params=pltpu.CompilerParams(use_tc_tiling_on_sc=False),
    )
    def agg(xt_hbm, src_hbm, dst_hbm, out_hbm, sidx, didx, rows, acc,
            g0, g1, i0, i1, s0, s1):
        c = lax.axis_index("c")
        s = lax.axis_index("s")
        gsems = (g0, g1)
        isems = (i0, i1)
        ssems = (s0, s1)

        row0 = s * _RPT
        pltpu.sync_copy(xt_hbm.at[c, pl.ds(row0, _RPT)],
                        acc.at[pl.ds(row0, _RPT)])
        plsc.subcore_barrier()

        ebase = s * _IROWS_PT

        def idx_descs(k, buf):
            irow = ebase + k * _K
            return (pltpu.make_async_copy(src_hbm.at[pl.ds(irow, _K)],
                                          sidx.at[buf], isems[buf]),
                    pltpu.make_async_copy(dst_hbm.at[pl.ds(irow, _K)],
                                          didx.at[buf], isems[buf]))

        def fire_gathers(buf):
            for j in range(_K):
                pltpu.async_copy(xt_hbm.at[c].at[sidx.at[buf, j]],
                                 rows.at[buf, pl.ds(j * 128, 128)],
                                 gsems[buf])

        def fire_scatters(buf):
            for j in range(_K):
                pltpu.async_copy(rows.at[buf, pl.ds(j * 128, 128)],
                                 acc.at[didx.at[buf, j]], ssems[buf],
                                 add=True)

        def drain(sem, buf):
            pltpu.make_async_copy(xt_hbm.at[0, pl.ds(0, _CHUNK)],
                                  rows.at[buf], sem).wait()

        da, db = idx_descs(0, 0)
        da.start()
        db.start()
        da.wait()
        db.wait()
        fire_gathers(0)

        def outer(i, carry):
            for b in range(2):
                k = i * 2 + b
                nb = 1 - b

                @pl.when(k >= 1)
                def _():
                    drain(ssems[nb], nb)

                @pl.when(k < _CPT - 1)
                def _():
                    d1, d2 = idx_descs(k + 1, nb)
                    d1.start()
                    d2.start()

                drain(gsems[b], b)
                fire_scatters(b)

                @pl.when(k < _CPT - 1)
                def _():
                    d1, d2 = idx_descs(k + 1, nb)
                    d1.wait()
                    d2.wait()
                    fire_gathers(nb)
            return carry

        lax.fori_loop(0, _CPT // 2, outer, 0)
        drain(ssems[1], 1)
        plsc.subcore_barrier()

        pltpu.sync_copy(acc.at[pl.ds(row0, _RPT)],
                        out_hbm.at[c, pl.ds(row0, _RPT)])

    return agg(xt, src2, dst2)


def _tc_mlp_stats(hp, A0, A1, b1t, W2e, b2t):
    do4 = W2e.shape[1]
    do = do4 // 4

    def body(h0_ref, h1_ref, a_ref, b1_ref, w2_ref, b2_ref,
             y_ref, st_ref, sacc):
        i = pl.program_id(0)
        t = jnp.maximum(
            jnp.dot(h0_ref[...], a_ref[0],
                    preferred_element_type=jnp.float32)
            + jnp.dot(h1_ref[...], a_ref[1],
                      preferred_element_type=jnp.float32)
            + b1_ref[0, :], 0.0)
        y = jnp.dot(t, w2_ref[...], preferred_element_type=jnp.float32) \
            + b2_ref[0, :]
        y_ref[...] = y
        rid = i * _BP + lax.broadcasted_iota(jnp.int32, (_BP, 1), 0)
        ym = jnp.where(rid < _NQR, y, 0.0)
        s1_4 = jnp.sum(ym, axis=0)
        s2_4 = jnp.sum(ym * ym, axis=0)
        s1 = (s1_4[0:do] + s1_4[do:2 * do]
              + s1_4[2 * do:3 * do] + s1_4[3 * do:4 * do])
        s2 = (s2_4[0:do] + s2_4[do:2 * do]
              + s2_4[2 * do:3 * do] + s2_4[3 * do:4 * do])
        upd = jnp.concatenate([s1[None, :], s2[None, :]], axis=0)
        prev = jnp.where(i == 0, jnp.zeros_like(upd), sacc[...])
        sacc[...] = prev + upd

        @pl.when(i == _NBP - 1)
        def _():
            st_ref[...] = sacc[...]

    return pl.pallas_call(
        body,
        grid=(_NBP,),
        in_specs=[
            pl.BlockSpec((_BP, 128), lambda i: (i, 0)),
            pl.BlockSpec((_BP, 128), lambda i: (i + _NBP, 0)),
            pl.BlockSpec((2, 128, 256), lambda i: (0, 0, 0)),
            pl.BlockSpec((1, 256), lambda i: (0, 0)),
            pl.BlockSpec((256, do4), lambda i: (0, 0)),
            pl.BlockSpec((1, do4), lambda i: (0, 0)),
        ],
        out_specs=[
            pl.BlockSpec((_BP, do4), lambda i: (i, 0)),
            pl.BlockSpec((2, do), lambda i: (0, 0)),
        ],
        out_shape=[
            jax.ShapeDtypeStruct((_NQ, do4), jnp.float32),
            jax.ShapeDtypeStruct((2, do), jnp.float32),
        ],
        scratch_shapes=[pltpu.VMEM((2, do), jnp.float32)],
        compiler_params=pltpu.CompilerParams(
            dimension_semantics=("arbitrary",)),
    )(hp, hp, jnp.stack([A0, A1]), b1t, W2e, b2t)


def _bn_coeffs(st_ref, g_ref, be_ref, reps):
    mu = st_ref[0, :] * (1.0 / _N)
    var = st_ref[1, :] * (1.0 / _N) - mu * mu
    scale = g_ref[0, :] * lax.rsqrt(var + _EPS)
    shift = be_ref[0, :] - mu * scale
    return (jnp.concatenate([scale] * reps),
            jnp.concatenate([shift] * reps))


def _tc_norm(yp, st, g, be):
    def body(y_ref, st_ref, g_ref, be_ref, o_ref):
        scale4, shift4 = _bn_coeffs(st_ref, g_ref, be_ref, 4)
        yn = jnp.maximum(y_ref[...] * scale4[None, :] + shift4[None, :], 0.0)
        for cc in range(2):
            o_ref[cc] = jnp.concatenate(
                [yn[:, 64 * r + 32 * cc: 64 * r + 32 * cc + 32]
                 for r in range(4)], axis=1)

    return pl.pallas_call(
        body,
        grid=(_NBP,),
        in_specs=[
            pl.BlockSpec((_BP, 256), lambda i: (i, 0)),
            pl.BlockSpec((2, 64), lambda i: (0, 0)),
            pl.BlockSpec((1, 64), lambda i: (0, 0)),
            pl.BlockSpec((1, 64), lambda i: (0, 0)),
        ],
        out_specs=pl.BlockSpec((2, _BP, 128), lambda i: (0, i, 0)),
        out_shape=jax.ShapeDtypeStruct((2, _NQ, 128), jnp.float32),
        compiler_params=pltpu.CompilerParams(
            dimension_semantics=("arbitrary",)),
    )(yp, st, g.reshape(1, -1), be.reshape(1, -1))


def _tc_norm_pool(yp, st, g, be, batchT):
    def body(y_ref, st_ref, g_ref, be_ref, b_ref, o_ref, pacc, cacc):
        i = pl.program_id(0)
        scale4, shift4 = _bn_coeffs(st_ref, g_ref, be_ref, 4)
        yn = jnp.maximum(y_ref[...] * scale4[None, :] + shift4[None, :], 0.0)
        gi = lax.broadcasted_iota(jnp.int32, (_G, _BP), 0)
        ps = jnp.zeros((_G, 32), jnp.float32)
        cs = jnp.zeros((_G, 1), jnp.float32)
        for r in range(4):
            oh = (b_ref[0, r:r + 1] == gi).astype(jnp.float32)
            ps = ps + jnp.dot(oh, yn[:, 32 * r:32 * r + 32],
                              preferred_element_type=jnp.float32)
            cs = cs + jnp.sum(oh, axis=1, keepdims=True)
        pprev = jnp.where(i == 0, jnp.zeros_like(ps), pacc[...])
        cprev = jnp.where(i == 0, jnp.zeros_like(cs), cacc[...])
        pacc[...] = pprev + ps
        cacc[...] = cprev + cs

        @pl.when(i == _NBP - 1)
        def _():
            o_ref[...] = pacc[...] / jnp.maximum(cacc[...], 1.0)

    return pl.pallas_call(
        body,
        grid=(_NBP,),
        in_specs=[
            pl.BlockSpec((_BP, 128), lambda i: (i, 0)),
            pl.BlockSpec((2, 32), lambda i: (0, 0)),
            pl.BlockSpec((1, 32), lambda i: (0, 0)),
            pl.BlockSpec((1, 32), lambda i: (0, 0)),
            pl.BlockSpec((1, 4, _BP), lambda i: (i, 0, 0)),
        ],
        out_specs=pl.BlockSpec((_G, 32), lambda i: (0, 0)),
        out_shape=jax.ShapeDtypeStruct((_G, 32), jnp.float32),
        scratch_shapes=[
            pltpu.VMEM((_G, 32), jnp.float32),
            pltpu.VMEM((_G, 1), jnp.float32),
        ],
        compiler_params=pltpu.CompilerParams(
            dimension_semantics=("arbitrary",)),
    )(yp, st, g.reshape(1, -1), be.reshape(1, -1), batchT)


def kernel(x, edge_index, batch,
           W1_0, b1_0, W2_0, b2_0, g_0, be_0,
           W1_1, b1_1, W2_1, b2_1, g_1, be_1,
           W1_2, b1_2, W2_2, b2_2, g_2, be_2):
    params = [(W1_0, b1_0, W2_0, b2_0, g_0, be_0),
              (W1_1, b1_1, W2_1, b2_1, g_1, be_1),
              (W1_2, b1_2, W2_2, b2_2, g_2, be_2)]

    npad = _EP - _E
    src_p = jnp.concatenate([edge_index[0],
                             jnp.zeros((npad,), jnp.int32)])
    dst_p = jnp.concatenate([edge_index[1],
                             _N + (jnp.arange(npad, dtype=jnp.int32)
                                   % (_NP - _N))])
    src2 = src_p.reshape(_EP // 128, 128)
    dst2 = dst_p.reshape(_EP // 128, 128)
    batchT = jnp.pad(batch, (0, _NP - _N),
                     constant_values=_G).reshape(_NBP, _BP, 4).transpose(0, 2, 1)

    xpad = jnp.pad(x, ((0, _NP - _N), (0, 0)))
    xtp = jnp.concatenate([xpad[:, :32].reshape(_NQ, 128),
                           xpad[:, 32:].reshape(_NQ, 128)], axis=0)

    ey4 = jnp.eye(4, dtype=jnp.float32)
    out = None
    for l in range(3):
        W1, b1, W2, b2, g, be = params[l]
        A0 = jnp.kron(ey4, W1[:32])
        A1 = jnp.kron(ey4, W1[32:])
        W2e = jnp.kron(ey4, W2)
        b1t = jnp.tile(b1, 4).reshape(1, -1)
        b2t = jnp.tile(b2, 4).reshape(1, -1)
        hh = _sc_agg(xtp.reshape(2, _NP, 32), src2, dst2)
        hp = hh.reshape(2 * _NQ, 128)
        yp, st = _tc_mlp_stats(hp, A0, A1, b1t, W2e, b2t)
        if l < 2:
            xtp = _tc_norm(yp, st, g, be).reshape(2 * _NQ, 128)
        else:
            out = _tc_norm_pool(yp, st, g, be, batchT)
    return out


# speedup vs baseline: 9.0636x; 1.0000x over previous
import functools

import jax
import jax.numpy as jnp
from jax import lax
from jax.experimental import pallas as pl
from jax.experimental.pallas import tpu as pltpu
from jax.experimental.pallas import tpu_sc as plsc

_N = 50000
_E = 800000
_G = 64
_NP = 50048
_RPT = _NP // 16
_NQ = _NP // 4
_NQR = _N // 4

_K = 3
_CHUNK = _K * 128
_CPT = 132
_EPT = _CPT * _CHUNK
_EP = 16 * _EPT
_IROWS_PT = _CPT * _K

_BP = 3128
_NBP = _NQ // _BP
_EPS = 1e-5


def _sc_agg(xt, src2, dst2):
    mesh = plsc.VectorSubcoreMesh(core_axis_name="c", subcore_axis_name="s")

    @functools.partial(
        pl.kernel,
        out_type=jax.ShapeDtypeStruct((2, _NP, 32), jnp.float32),
        mesh=mesh,
        scratch_types=[
            pltpu.VMEM((2, _CHUNK), jnp.int32),
            pltpu.VMEM((2, _CHUNK), jnp.int32),
            pltpu.VMEM((2, _CHUNK, 32), jnp.float32),
            pltpu.VMEM_SHARED((_NP, 32), jnp.float32),
            pltpu.SemaphoreType.DMA,
            pltpu.SemaphoreType.DMA,
            pltpu.SemaphoreType.DMA,
            pltpu.SemaphoreType.DMA,
            pltpu.SemaphoreType.DMA,
            pltpu.SemaphoreType.DMA,
        ],
        compiler_params=pltpu.CompilerParams(use_tc_tiling_on_sc=False),
    )
    def agg(xt_hbm, src_hbm, dst_hbm, out_hbm, sidx, didx, rows, acc,
            g0, g1, i0, i1, s0, s1):
        c = lax.axis_index("c")
        s = lax.axis_index("s")
        gsems = (g0, g1)
        isems = (i0, i1)
        ssems = (s0, s1)

        row0 = s * _RPT
        pltpu.sync_copy(xt_hbm.at[c, pl.ds(row0, _RPT)],
                        acc.at[pl.ds(row0, _RPT)])
        plsc.subcore_barrier()

        ebase = s * _IROWS_PT

        def idx_descs(k, buf):
            e0 = ebase * 128 + k * _CHUNK
            return (pltpu.make_async_copy(src_hbm.at[pl.ds(e0, _CHUNK)],
                                          sidx.at[buf], isems[buf]),
                    pltpu.make_async_copy(dst_hbm.at[pl.ds(e0, _CHUNK)],
                                          didx.at[buf], isems[buf]))

        def fire_gathers(buf):
            pltpu.async_copy(xt_hbm.at[c].at[sidx.at[buf]],
                             rows.at[buf], gsems[buf])

        def fire_scatters(buf):
            pltpu.async_copy(rows.at[buf], acc.at[didx.at[buf]],
                             ssems[buf], add=True)

        def drain(sem, buf):
            pltpu.make_async_copy(xt_hbm.at[0, pl.ds(0, _CHUNK)],
                                  rows.at[buf], sem).wait()

        da, db = idx_descs(0, 0)
        da.start()
        db.start()
        da.wait()
        db.wait()
        fire_gathers(0)

        def outer(i, carry):
            for b in range(2):
                k = i * 2 + b
                nb = 1 - b

                @pl.when(k >= 1)
                def _():
                    drain(ssems[nb], nb)

                @pl.when(k < _CPT - 1)
                def _():
                    d1, d2 = idx_descs(k + 1, nb)
                    d1.start()
                    d2.start()

                drain(gsems[b], b)
                fire_scatters(b)

                @pl.when(k < _CPT - 1)
                def _():
                    d1, d2 = idx_descs(k + 1, nb)
                    d1.wait()
                    d2.wait()
                    fire_gathers(nb)
            return carry

        lax.fori_loop(0, _CPT // 2, outer, 0)
        drain(ssems[1], 1)
        plsc.subcore_barrier()

        pltpu.sync_copy(acc.at[pl.ds(row0, _RPT)],
                        out_hbm.at[c, pl.ds(row0, _RPT)])

    return agg(xt, src2, dst2)


def _tc_mlp_stats(hp, A0, A1, b1t, W2e, b2t):
    do4 = W2e.shape[1]
    do = do4 // 4

    def body(h0_ref, h1_ref, a_ref, b1_ref, w2_ref, b2_ref,
             y_ref, st_ref, sacc):
        i = pl.program_id(0)
        t = jnp.maximum(
            jnp.dot(h0_ref[...], a_ref[0],
                    preferred_element_type=jnp.float32)
            + jnp.dot(h1_ref[...], a_ref[1],
                      preferred_element_type=jnp.float32)
            + b1_ref[0, :], 0.0)
        y = jnp.dot(t, w2_ref[...], preferred_element_type=jnp.float32) \
            + b2_ref[0, :]
        y_ref[...] = y
        rid = i * _BP + lax.broadcasted_iota(jnp.int32, (_BP, 1), 0)
        ym = jnp.where(rid < _NQR, y, 0.0)
        s1_4 = jnp.sum(ym, axis=0)
        s2_4 = jnp.sum(ym * ym, axis=0)
        s1 = (s1_4[0:do] + s1_4[do:2 * do]
              + s1_4[2 * do:3 * do] + s1_4[3 * do:4 * do])
        s2 = (s2_4[0:do] + s2_4[do:2 * do]
              + s2_4[2 * do:3 * do] + s2_4[3 * do:4 * do])
        upd = jnp.concatenate([s1[None, :], s2[None, :]], axis=0)
        prev = jnp.where(i == 0, jnp.zeros_like(upd), sacc[...])
        sacc[...] = prev + upd

        @pl.when(i == _NBP - 1)
        def _():
            st_ref[...] = sacc[...]

    return pl.pallas_call(
        body,
        grid=(_NBP,),
        in_specs=[
            pl.BlockSpec((_BP, 128), lambda i: (i, 0)),
            pl.BlockSpec((_BP, 128), lambda i: (i + _NBP, 0)),
            pl.BlockSpec((2, 128, 256), lambda i: (0, 0, 0)),
            pl.BlockSpec((1, 256), lambda i: (0, 0)),
            pl.BlockSpec((256, do4), lambda i: (0, 0)),
            pl.BlockSpec((1, do4), lambda i: (0, 0)),
        ],
        out_specs=[
            pl.BlockSpec((_BP, do4), lambda i: (i, 0)),
            pl.BlockSpec((2, do), lambda i: (0, 0)),
        ],
        out_shape=[
            jax.ShapeDtypeStruct((_NQ, do4), jnp.float32),
            jax.ShapeDtypeStruct((2, do), jnp.float32),
        ],
        scratch_shapes=[pltpu.VMEM((2, do), jnp.float32)],
        compiler_params=pltpu.CompilerParams(
            dimension_semantics=("arbitrary",)),
    )(hp, hp, jnp.stack([A0, A1]), b1t, W2e, b2t)


def _bn_coeffs(st_ref, g_ref, be_ref, reps):
    mu = st_ref[0, :] * (1.0 / _N)
    var = st_ref[1, :] * (1.0 / _N) - mu * mu
    scale = g_ref[0, :] * lax.rsqrt(var + _EPS)
    shift = be_ref[0, :] - mu * scale
    return (jnp.concatenate([scale] * reps),
            jnp.concatenate([shift] * reps))


def _tc_norm(yp, st, g, be):
    def body(y_ref, st_ref, g_ref, be_ref, o_ref):
        scale4, shift4 = _bn_coeffs(st_ref, g_ref, be_ref, 4)
        yn = jnp.maximum(y_ref[...] * scale4[None, :] + shift4[None, :], 0.0)
        for cc in range(2):
            o_ref[cc] = jnp.concatenate(
                [yn[:, 64 * r + 32 * cc: 64 * r + 32 * cc + 32]
                 for r in range(4)], axis=1)

    return pl.pallas_call(
        body,
        grid=(_NBP,),
        in_specs=[
            pl.BlockSpec((_BP, 256), lambda i: (i, 0)),
            pl.BlockSpec((2, 64), lambda i: (0, 0)),
            pl.BlockSpec((1, 64), lambda i: (0, 0)),
            pl.BlockSpec((1, 64), lambda i: (0, 0)),
        ],
        out_specs=pl.BlockSpec((2, _BP, 128), lambda i: (0, i, 0)),
        out_shape=jax.ShapeDtypeStruct((2, _NQ, 128), jnp.float32),
        compiler_params=pltpu.CompilerParams(
            dimension_semantics=("arbitrary",)),
    )(yp, st, g.reshape(1, -1), be.reshape(1, -1))


def _tc_norm_pool(yp, st, g, be, batchT):
    def body(y_ref, st_ref, g_ref, be_ref, b_ref, o_ref, pacc, cacc):
        i = pl.program_id(0)
        scale4, shift4 = _bn_coeffs(st_ref, g_ref, be_ref, 4)
        yn = jnp.maximum(y_ref[...] * scale4[None, :] + shift4[None, :], 0.0)
        gi = lax.broadcasted_iota(jnp.int32, (_G, _BP), 0)
        ps = jnp.zeros((_G, 32), jnp.float32)
        cs = jnp.zeros((_G, 1), jnp.float32)
        for r in range(4):
            oh = (b_ref[0, r:r + 1] == gi).astype(jnp.float32)
            ps = ps + jnp.dot(oh, yn[:, 32 * r:32 * r + 32],
                              preferred_element_type=jnp.float32)
            cs = cs + jnp.sum(oh, axis=1, keepdims=True)
        pprev = jnp.where(i == 0, jnp.zeros_like(ps), pacc[...])
        cprev = jnp.where(i == 0, jnp.zeros_like(cs), cacc[...])
        pacc[...] = pprev + ps
        cacc[...] = cprev + cs

        @pl.when(i == _NBP - 1)
        def _():
            o_ref[...] = pacc[...] / jnp.maximum(cacc[...], 1.0)

    return pl.pallas_call(
        body,
        grid=(_NBP,),
        in_specs=[
            pl.BlockSpec((_BP, 128), lambda i: (i, 0)),
            pl.BlockSpec((2, 32), lambda i: (0, 0)),
            pl.BlockSpec((1, 32), lambda i: (0, 0)),
            pl.BlockSpec((1, 32), lambda i: (0, 0)),
            pl.BlockSpec((1, 4, _BP), lambda i: (i, 0, 0)),
        ],
        out_specs=pl.BlockSpec((_G, 32), lambda i: (0, 0)),
        out_shape=jax.ShapeDtypeStruct((_G, 32), jnp.float32),
        scratch_shapes=[
            pltpu.VMEM((_G, 32), jnp.float32),
            pltpu.VMEM((_G, 1), jnp.float32),
        ],
        compiler_params=pltpu.CompilerParams(
            dimension_semantics=("arbitrary",)),
    )(yp, st, g.reshape(1, -1), be.reshape(1, -1), batchT)


def kernel(x, edge_index, batch,
           W1_0, b1_0, W2_0, b2_0, g_0, be_0,
           W1_1, b1_1, W2_1, b2_1, g_1, be_1,
           W1_2, b1_2, W2_2, b2_2, g_2, be_2):
    params = [(W1_0, b1_0, W2_0, b2_0, g_0, be_0),
              (W1_1, b1_1, W2_1, b2_1, g_1, be_1),
              (W1_2, b1_2, W2_2, b2_2, g_2, be_2)]

    npad = _EP - _E
    src_p = jnp.concatenate([edge_index[0],
                             jnp.zeros((npad,), jnp.int32)])
    dst_p = jnp.concatenate([edge_index[1],
                             _N + (jnp.arange(npad, dtype=jnp.int32)
                                   % (_NP - _N))])
    src2 = src_p
    dst2 = dst_p
    batchT = jnp.pad(batch, (0, _NP - _N),
                     constant_values=_G).reshape(_NBP, _BP, 4).transpose(0, 2, 1)

    xpad = jnp.pad(x, ((0, _NP - _N), (0, 0)))
    xtp = jnp.concatenate([xpad[:, :32].reshape(_NQ, 128),
                           xpad[:, 32:].reshape(_NQ, 128)], axis=0)

    ey4 = jnp.eye(4, dtype=jnp.float32)
    out = None
    for l in range(3):
        W1, b1, W2, b2, g, be = params[l]
        A0 = jnp.kron(ey4, W1[:32])
        A1 = jnp.kron(ey4, W1[32:])
        W2e = jnp.kron(ey4, W2)
        b1t = jnp.tile(b1, 4).reshape(1, -1)
        b2t = jnp.tile(b2, 4).reshape(1, -1)
        hh = _sc_agg(xtp.reshape(2, _NP, 32), src2, dst2)
        hp = hh.reshape(2 * _NQ, 128)
        yp, st = _tc_mlp_stats(hp, A0, A1, b1t, W2e, b2t)
        if l < 2:
            xtp = _tc_norm(yp, st, g, be).reshape(2 * _NQ, 128)
        else:
            out = _tc_norm_pool(yp, st, g, be, batchT)
    return out
